# baseline (speedup 1.0000x reference)
import functools

import jax
import jax.numpy as jnp
from jax import lax
from jax.experimental import pallas as pl
from jax.experimental.pallas import tpu as pltpu
from jax.experimental.pallas import tpu_sc as plsc

_D = 16
_S = 26
_B = 16384
_MAXN = 1.0 - 0.001
_NC, _NS = 2, 16
_NW = _NC * _NS
_B_TOT = _B * _S
_R_PER_W = _B_TOT // _NW
_BW = _B // _NW
_CB = 64
_CHUNK = _CB * _S
_NCHUNK = _BW // _CB

_mesh = plsc.VectorSubcoreMesh(
    core_axis_name="c", subcore_axis_name="s",
    num_cores=_NC, num_subcores=_NS)


def _rsqrt(n2):
    i = plsc.bitcast(n2, jnp.int32)
    i = jnp.int32(0x5F3759DF) - (i >> 1)
    y = plsc.bitcast(i, jnp.float32)
    for _ in range(3):
        y = y * (1.5 - 0.5 * n2 * y * y)
    return y


@functools.partial(
    pl.kernel,
    out_type=jax.ShapeDtypeStruct((_S * _D, _B), jnp.float32),
    mesh=_mesh,
    compiler_params=pltpu.CompilerParams(
        needs_layout_passes=False, use_tc_tiling_on_sc=False),
    scratch_types=[
        pltpu.VMEM((2, 1, _CHUNK), jnp.int32),
        pltpu.VMEM((2, _CHUNK, _D), jnp.float32),
        pltpu.VMEM((2, _S * _D, _CB), jnp.float32),
        pltpu.SemaphoreType.DMA,
        pltpu.SemaphoreType.DMA,
        pltpu.SemaphoreType.DMA,
        pltpu.SemaphoreType.DMA,
    ],
)
def _emb_lookup(x_hbm, w_hbm, out_hbm, idx_v, rows_v, trans_v,
                semg0, semg1, semo0, semo1):
    wid = lax.axis_index("s") * _NC + lax.axis_index("c")
    iota16 = lax.iota(jnp.int32, 16)
    semg = (semg0, semg1)
    semo = (semo0, semo1)

    def start_gather(c):
        b = c & 1
        off = wid * _R_PER_W + c * _CHUNK
        pltpu.sync_copy(x_hbm.at[pl.ds(off, _CHUNK)], idx_v.at[b, 0])
        return pltpu.async_copy(w_hbm.at[idx_v.at[b, 0]], rows_v.at[b],
                                semg[b])

    def compute(c):
        b = c & 1

        def group(g, carry):
            j = g * 16 + iota16
            bl = j // _S
            s16 = (j - bl * _S) * _D
            cols = []
            n2 = jnp.zeros((16,), jnp.float32)
            for d in range(_D):
                dsplat = jnp.full((16,), d, jnp.int32)
                col = plsc.load_gather(rows_v.at[b], [j, dsplat])
                n2 = n2 + col * col
                cols.append(col)
            scale = jnp.where(n2 > _MAXN * _MAXN,
                              (-_MAXN) * _rsqrt(n2), -1.0)
            for d in range(_D):
                plsc.store_scatter(trans_v.at[b], [s16 + d, bl],
                                   cols[d] * scale)
            return carry

        lax.fori_loop(0, _CHUNK // 16, group, 0)

    def start_out(c):
        b = c & 1
        b0 = wid * _BW + c * _CB
        return pltpu.async_copy(trans_v.at[b], out_hbm.at[:, pl.ds(b0, _CB)],
                                semo[b])

    hg = [None, None]
    ho = [None, None]
    hg[0] = start_gather(0)
    for c in range(_NCHUNK):
        b = c & 1
        if c + 1 < _NCHUNK:
            hg[1 - b] = start_gather(c + 1)
        hg[b].wait()
        if ho[b] is not None:
            ho[b].wait()
        compute(c)
        ho[b] = start_out(c)
    ho[0].wait()
    ho[1].wait()


def kernel(x, W):
    xf = x.reshape(-1).astype(jnp.int32)
    out = _emb_lookup(xf, -W)
    return out.reshape(_S, _D, _B).transpose(2, 0, 1)


# speedup vs baseline: 1.0524x; 1.0524x over previous
import functools

import jax
import jax.numpy as jnp
from jax import lax
from jax.experimental import pallas as pl
from jax.experimental.pallas import tpu as pltpu
from jax.experimental.pallas import tpu_sc as plsc

_D = 16
_S = 26
_B = 16384
_MAXN = 1.0 - 0.001
_NC, _NS = 2, 16
_NW = _NC * _NS
_B_TOT = _B * _S
_R_PER_W = _B_TOT // _NW
_BW = _B // _NW
_CB = 64
_CHUNK = _CB * _S
_NCHUNK = _BW // _CB

_mesh = plsc.VectorSubcoreMesh(
    core_axis_name="c", subcore_axis_name="s",
    num_cores=_NC, num_subcores=_NS)


def _rsqrt(n2):
    i = plsc.bitcast(n2, jnp.int32)
    i = jnp.int32(0x5F3759DF) - (i >> 1)
    y = plsc.bitcast(i, jnp.float32)
    for _ in range(3):
        y = y * (1.5 - 0.5 * n2 * y * y)
    return y


@functools.partial(
    pl.kernel,
    out_type=jax.ShapeDtypeStruct((_S * _D, _B), jnp.float32),
    mesh=_mesh,
    compiler_params=pltpu.CompilerParams(
        needs_layout_passes=False, use_tc_tiling_on_sc=False),
    scratch_types=[
        pltpu.VMEM((2, 1, _CHUNK), jnp.int32),
        pltpu.VMEM((2, _CHUNK, _D), jnp.float32),
        pltpu.VMEM((2, _S * _D, _CB), jnp.float32),
        pltpu.SemaphoreType.DMA,
        pltpu.SemaphoreType.DMA,
        pltpu.SemaphoreType.DMA,
        pltpu.SemaphoreType.DMA,
    ],
)
def _emb_lookup(x_hbm, w_hbm, out_hbm, idx_v, rows_v, trans_v,
                semg0, semg1, semo0, semo1):
    wid = lax.axis_index("s") * _NC + lax.axis_index("c")
    iota16 = lax.iota(jnp.int32, 16)
    semg = (semg0, semg1)
    semo = (semo0, semo1)

    def start_gather(c):
        b = c & 1
        off = wid * _R_PER_W + c * _CHUNK
        pltpu.sync_copy(x_hbm.at[pl.ds(off, _CHUNK)], idx_v.at[b, 0])
        return pltpu.async_copy(w_hbm.at[idx_v.at[b, 0]], rows_v.at[b],
                                semg[b])

    def compute(c):
        b = c & 1

        def group(g, carry):
            j = g * 16 + iota16
            bl = j // _S
            s16 = (j - bl * _S) * _D
            cols = []
            n2 = jnp.zeros((16,), jnp.float32)
            for d in range(_D):
                dsplat = jnp.full((16,), d, jnp.int32)
                col = plsc.load_gather(rows_v.at[b], [j, dsplat])
                n2 = n2 + col * col
                cols.append(col)
            scale = jnp.where(n2 > _MAXN * _MAXN, _MAXN * _rsqrt(n2), 1.0)
            for d in range(_D):
                plsc.store_scatter(trans_v.at[b], [s16 + d, bl],
                                   cols[d] * scale)
            return carry

        lax.fori_loop(0, _CHUNK // 16, group, 0)

    def start_out(c):
        b = c & 1
        b0 = wid * _BW + c * _CB
        return pltpu.async_copy(trans_v.at[b], out_hbm.at[:, pl.ds(b0, _CB)],
                                semo[b])

    hg = [None, None]
    ho = [None, None]
    hg[0] = start_gather(0)
    for c in range(_NCHUNK):
        b = c & 1
        if c + 1 < _NCHUNK:
            hg[1 - b] = start_gather(c + 1)
        hg[b].wait()
        if ho[b] is not None:
            ho[b].wait()
        compute(c)
        ho[b] = start_out(c)
    ho[0].wait()
    ho[1].wait()


def kernel(x, W):
    xf = x.reshape(-1).astype(jnp.int32)
    out = _emb_lookup(xf, W)
    return out.reshape(_S, _D, _B).transpose(2, 0, 1)
